# baseline (speedup 1.0000x reference)
import jax
import jax.numpy as jnp
from jax import lax
from jax.experimental import pallas as pl
from jax.experimental.pallas import tpu as pltpu

_TOKENS = 8192
_HIDDEN = 2048
_EXPERTS = 16
_TB = 1024
_NC = _TOKENS // _TB


def _route_chunk(lg):
    lt = lg.T
    l0 = lt[0:1, :]
    l1 = lt[1:2, :]
    gt = l1 > l0
    m1 = jnp.where(gt, l1, l0)
    m2 = jnp.where(gt, l0, l1)
    i1 = jnp.where(gt, jnp.int32(1), jnp.int32(0))
    i2 = jnp.where(gt, jnp.int32(0), jnp.int32(1))
    for e in range(2, _EXPERTS):
        le = lt[e:e + 1, :]
        ev = jnp.int32(e)
        gt1 = le > m1
        gt2 = le > m2
        i2 = jnp.where(gt1, i1, jnp.where(gt2, ev, i2))
        m2 = jnp.where(gt1, m1, jnp.where(gt2, le, m2))
        i1 = jnp.where(gt1, ev, i1)
        m1 = jnp.where(gt1, le, m1)
    r = jnp.exp(m2 - m1)
    s = r + jnp.float32(1.0)
    w1 = jnp.float32(1.0) / s
    w2 = r / s
    wpair = jnp.concatenate([w1, w2], axis=0).T
    ipair = jnp.concatenate([i1, i2], axis=0).T
    eio = lax.broadcasted_iota(jnp.int32, (_EXPERTS, _TB), 0)
    mk1 = (eio == i1).astype(jnp.int32)
    mk2 = (eio == i2).astype(jnp.int32)
    mask = jnp.stack([mk1, mk2], axis=1)
    return wpair, ipair, mask


def _body(x_hbm, w_ref, b_ref, lg_hbm, wp_hbm, ip_hbm, mk_hbm):
    def scoped(xb, lgb, wpb, ipb, mkb, in_sem, out_sem):
        def copy_in(i, slot):
            return pltpu.make_async_copy(
                x_hbm.at[pl.ds(i * _TB, _TB), :], xb.at[slot], in_sem.at[slot])

        def out_copies(i, slot):
            return (
                pltpu.make_async_copy(
                    lgb.at[slot], lg_hbm.at[pl.ds(i * _TB, _TB), :],
                    out_sem.at[0, slot]),
                pltpu.make_async_copy(
                    wpb.at[slot], wp_hbm.at[pl.ds(i * _TB, _TB), :],
                    out_sem.at[1, slot]),
                pltpu.make_async_copy(
                    ipb.at[slot], ip_hbm.at[pl.ds(i * _TB, _TB), :],
                    out_sem.at[2, slot]),
                pltpu.make_async_copy(
                    mkb.at[slot], mk_hbm.at[:, :, pl.ds(i * _TB, _TB)],
                    out_sem.at[3, slot]),
            )

        copy_in(0, 0).start()
        for i in range(_NC):
            slot = i % 2
            if i + 1 < _NC:
                copy_in(i + 1, (i + 1) % 2).start()
            copy_in(i, slot).wait()
            if i >= 2:
                for c in out_copies(i - 2, slot):
                    c.wait()
            lg = lax.dot_general(
                xb[slot], w_ref[...],
                (((1,), (1,)), ((), ())),
                preferred_element_type=jnp.float32,
            ) + b_ref[...]
            wpair, ipair, mask = _route_chunk(lg)
            lgb[slot] = lg
            wpb[slot] = wpair
            ipb[slot] = ipair
            mkb[slot] = mask
            for c in out_copies(i, slot):
                c.start()
        for i in (_NC - 2, _NC - 1):
            for c in out_copies(i, i % 2):
                c.wait()

    pl.run_scoped(
        scoped,
        pltpu.VMEM((2, _TB, _HIDDEN), jnp.float32),
        pltpu.VMEM((2, _TB, _EXPERTS), jnp.float32),
        pltpu.VMEM((2, _TB, 2), jnp.float32),
        pltpu.VMEM((2, _TB, 2), jnp.int32),
        pltpu.VMEM((2, _EXPERTS, 2, _TB), jnp.int32),
        pltpu.SemaphoreType.DMA((2,)),
        pltpu.SemaphoreType.DMA((4, 2)),
    )


_fused = pl.pallas_call(
    _body,
    in_specs=[
        pl.BlockSpec(memory_space=pl.ANY),
        pl.BlockSpec(memory_space=pltpu.MemorySpace.VMEM),
        pl.BlockSpec(memory_space=pltpu.MemorySpace.VMEM),
    ],
    out_specs=[
        pl.BlockSpec(memory_space=pl.ANY),
        pl.BlockSpec(memory_space=pl.ANY),
        pl.BlockSpec(memory_space=pl.ANY),
        pl.BlockSpec(memory_space=pl.ANY),
    ],
    out_shape=[
        jax.ShapeDtypeStruct((_TOKENS, _EXPERTS), jnp.float32),
        jax.ShapeDtypeStruct((_TOKENS, 2), jnp.float32),
        jax.ShapeDtypeStruct((_TOKENS, 2), jnp.int32),
        jax.ShapeDtypeStruct((_EXPERTS, 2, _TOKENS), jnp.int32),
    ],
)


def kernel(x, W, b):
    logits, router_weight, select_idx, expert_mask = _fused(
        x, W, b.reshape(1, _EXPERTS))
    return (logits, router_weight, select_idx, expert_mask)


# speedup vs baseline: 1.0529x; 1.0529x over previous
import jax
import jax.numpy as jnp
from jax import lax
from jax.experimental import pallas as pl
from jax.experimental.pallas import tpu as pltpu

_TOKENS = 8192
_HIDDEN = 2048
_EXPERTS = 16
_TB = 1024
_NC = _TOKENS // _TB
_NBUF = 4
_NOB = 2


def _route_chunk(lg):
    lt = lg.T
    l0 = lt[0:1, :]
    l1 = lt[1:2, :]
    gt = l1 > l0
    m1 = jnp.where(gt, l1, l0)
    m2 = jnp.where(gt, l0, l1)
    i1 = jnp.where(gt, jnp.int32(1), jnp.int32(0))
    i2 = jnp.where(gt, jnp.int32(0), jnp.int32(1))
    for e in range(2, _EXPERTS):
        le = lt[e:e + 1, :]
        ev = jnp.int32(e)
        gt1 = le > m1
        gt2 = le > m2
        i2 = jnp.where(gt1, i1, jnp.where(gt2, ev, i2))
        m2 = jnp.where(gt1, m1, jnp.where(gt2, le, m2))
        i1 = jnp.where(gt1, ev, i1)
        m1 = jnp.where(gt1, le, m1)
    r = jnp.exp(m2 - m1)
    s = r + jnp.float32(1.0)
    w1 = jnp.float32(1.0) / s
    w2 = r / s
    wpair = jnp.concatenate([w1, w2], axis=0).T
    ipair = jnp.concatenate([i1, i2], axis=0).T
    eio = lax.broadcasted_iota(jnp.int32, (_EXPERTS, _TB), 0)
    mk1 = (eio == i1).astype(jnp.int32)
    mk2 = (eio == i2).astype(jnp.int32)
    return wpair, ipair, mk1, mk2


def _body(x_hbm, w_ref, b_ref, lg_hbm, wp_hbm, ip_hbm, mk_hbm):
    def scoped(xb, lgb, wpb, ipb, mkb, in_sem, out_sem):
        def copy_in(i):
            return pltpu.make_async_copy(
                x_hbm.at[pl.ds(i * _TB, _TB), :], xb.at[i % _NBUF],
                in_sem.at[i % _NBUF])

        def out_copies(i):
            slot = i % _NOB
            return (
                pltpu.make_async_copy(
                    lgb.at[slot], lg_hbm.at[pl.ds(i * _TB, _TB), :],
                    out_sem.at[0, slot]),
                pltpu.make_async_copy(
                    wpb.at[slot], wp_hbm.at[pl.ds(i * _TB, _TB), :],
                    out_sem.at[1, slot]),
                pltpu.make_async_copy(
                    ipb.at[slot], ip_hbm.at[pl.ds(i * _TB, _TB), :],
                    out_sem.at[2, slot]),
                pltpu.make_async_copy(
                    mkb.at[slot], mk_hbm.at[:, :, pl.ds(i * _TB, _TB)],
                    out_sem.at[3, slot]),
            )

        for i in range(_NBUF):
            copy_in(i).start()
        for i in range(_NC):
            slot = i % _NOB
            copy_in(i).wait()
            if i >= _NOB:
                for c in out_copies(i - _NOB):
                    c.wait()
            lg = lax.dot_general(
                xb[i % _NBUF], w_ref[...],
                (((1,), (1,)), ((), ())),
                preferred_element_type=jnp.float32,
            ) + b_ref[...]
            if i + _NBUF < _NC:
                copy_in(i + _NBUF).start()
            wpair, ipair, mk1, mk2 = _route_chunk(lg)
            lgb[slot] = lg
            wpb[slot] = wpair
            ipb[slot] = ipair
            mkb[slot, :, 0, :] = mk1
            mkb[slot, :, 1, :] = mk2
            for c in out_copies(i):
                c.start()
        for i in (_NC - 2, _NC - 1):
            for c in out_copies(i):
                c.wait()

    pl.run_scoped(
        scoped,
        pltpu.VMEM((_NBUF, _TB, _HIDDEN), jnp.float32),
        pltpu.VMEM((_NOB, _TB, _EXPERTS), jnp.float32),
        pltpu.VMEM((_NOB, _TB, 2), jnp.float32),
        pltpu.VMEM((_NOB, _TB, 2), jnp.int32),
        pltpu.VMEM((_NOB, _EXPERTS, 2, _TB), jnp.int32),
        pltpu.SemaphoreType.DMA((_NBUF,)),
        pltpu.SemaphoreType.DMA((4, _NOB)),
    )


_fused = pl.pallas_call(
    _body,
    in_specs=[
        pl.BlockSpec(memory_space=pl.ANY),
        pl.BlockSpec(memory_space=pltpu.MemorySpace.VMEM),
        pl.BlockSpec(memory_space=pltpu.MemorySpace.VMEM),
    ],
    out_specs=[
        pl.BlockSpec(memory_space=pl.ANY),
        pl.BlockSpec(memory_space=pl.ANY),
        pl.BlockSpec(memory_space=pl.ANY),
        pl.BlockSpec(memory_space=pl.ANY),
    ],
    out_shape=[
        jax.ShapeDtypeStruct((_TOKENS, _EXPERTS), jnp.float32),
        jax.ShapeDtypeStruct((_TOKENS, 2), jnp.float32),
        jax.ShapeDtypeStruct((_TOKENS, 2), jnp.int32),
        jax.ShapeDtypeStruct((_EXPERTS, 2, _TOKENS), jnp.int32),
    ],
)


def kernel(x, W, b):
    logits, router_weight, select_idx, expert_mask = _fused(
        x, W, b.reshape(1, _EXPERTS))
    return (logits, router_weight, select_idx, expert_mask)


# speedup vs baseline: 1.0571x; 1.0040x over previous
import jax
import jax.numpy as jnp
from jax import lax
from jax.experimental import pallas as pl
from jax.experimental.pallas import tpu as pltpu

_TOKENS = 8192
_HIDDEN = 2048
_EXPERTS = 16
_TB = 1024
_NC = _TOKENS // _TB
_NBUF = 4
_NOB = 2


def _route_chunk(lg):
    lt = lg.T
    eio = lax.broadcasted_iota(jnp.int32, (_EXPERTS, _TB), 0)
    big_i = jnp.int32(_EXPERTS)
    m1 = jnp.max(lt, axis=0, keepdims=True)
    i1 = jnp.min(jnp.where(lt == m1, eio, big_i), axis=0, keepdims=True)
    sel1 = eio == i1
    ltm = jnp.where(sel1, jnp.float32(-jnp.inf), lt)
    m2 = jnp.max(ltm, axis=0, keepdims=True)
    i2 = jnp.min(jnp.where(ltm == m2, eio, big_i), axis=0, keepdims=True)
    r = jnp.exp(m2 - m1)
    s = r + jnp.float32(1.0)
    w1 = jnp.float32(1.0) / s
    w2 = r / s
    wpair = jnp.concatenate([w1, w2], axis=0).T
    ipair = jnp.concatenate([i1, i2], axis=0).T
    mk1 = sel1.astype(jnp.int32)
    mk2 = (eio == i2).astype(jnp.int32)
    return wpair, ipair, mk1, mk2


def _body(x_hbm, w_ref, b_ref, lg_hbm, wp_hbm, ip_hbm, mk_hbm):
    def scoped(xb, lgb, wpb, ipb, mkb, in_sem, out_sem):
        def copy_in(i):
            return pltpu.make_async_copy(
                x_hbm.at[pl.ds(i * _TB, _TB), :], xb.at[i % _NBUF],
                in_sem.at[i % _NBUF])

        def out_copies(i):
            slot = i % _NOB
            return (
                pltpu.make_async_copy(
                    lgb.at[slot], lg_hbm.at[pl.ds(i * _TB, _TB), :],
                    out_sem.at[0, slot]),
                pltpu.make_async_copy(
                    wpb.at[slot], wp_hbm.at[pl.ds(i * _TB, _TB), :],
                    out_sem.at[1, slot]),
                pltpu.make_async_copy(
                    ipb.at[slot], ip_hbm.at[pl.ds(i * _TB, _TB), :],
                    out_sem.at[2, slot]),
                pltpu.make_async_copy(
                    mkb.at[slot], mk_hbm.at[:, :, pl.ds(i * _TB, _TB)],
                    out_sem.at[3, slot]),
            )

        for i in range(_NBUF):
            copy_in(i).start()
        for i in range(_NC):
            slot = i % _NOB
            copy_in(i).wait()
            if i >= _NOB:
                for c in out_copies(i - _NOB):
                    c.wait()
            lg = lax.dot_general(
                xb[i % _NBUF], w_ref[...],
                (((1,), (1,)), ((), ())),
                preferred_element_type=jnp.float32,
            ) + b_ref[...]
            if i + _NBUF < _NC:
                copy_in(i + _NBUF).start()
            wpair, ipair, mk1, mk2 = _route_chunk(lg)
            lgb[slot] = lg
            wpb[slot] = wpair
            ipb[slot] = ipair
            mkb[slot, :, 0, :] = mk1
            mkb[slot, :, 1, :] = mk2
            for c in out_copies(i):
                c.start()
        for i in (_NC - 2, _NC - 1):
            for c in out_copies(i):
                c.wait()

    pl.run_scoped(
        scoped,
        pltpu.VMEM((_NBUF, _TB, _HIDDEN), jnp.float32),
        pltpu.VMEM((_NOB, _TB, _EXPERTS), jnp.float32),
        pltpu.VMEM((_NOB, _TB, 2), jnp.float32),
        pltpu.VMEM((_NOB, _TB, 2), jnp.int32),
        pltpu.VMEM((_NOB, _EXPERTS, 2, _TB), jnp.int32),
        pltpu.SemaphoreType.DMA((_NBUF,)),
        pltpu.SemaphoreType.DMA((4, _NOB)),
    )


_fused = pl.pallas_call(
    _body,
    in_specs=[
        pl.BlockSpec(memory_space=pl.ANY),
        pl.BlockSpec(memory_space=pltpu.MemorySpace.VMEM),
        pl.BlockSpec(memory_space=pltpu.MemorySpace.VMEM),
    ],
    out_specs=[
        pl.BlockSpec(memory_space=pl.ANY),
        pl.BlockSpec(memory_space=pl.ANY),
        pl.BlockSpec(memory_space=pl.ANY),
        pl.BlockSpec(memory_space=pl.ANY),
    ],
    out_shape=[
        jax.ShapeDtypeStruct((_TOKENS, _EXPERTS), jnp.float32),
        jax.ShapeDtypeStruct((_TOKENS, 2), jnp.float32),
        jax.ShapeDtypeStruct((_TOKENS, 2), jnp.int32),
        jax.ShapeDtypeStruct((_EXPERTS, 2, _TOKENS), jnp.int32),
    ],
)


def kernel(x, W, b):
    logits, router_weight, select_idx, expert_mask = _fused(
        x, W, b.reshape(1, _EXPERTS))
    return (logits, router_weight, select_idx, expert_mask)


# speedup vs baseline: 1.0716x; 1.0137x over previous
import jax
import jax.numpy as jnp
from jax import lax
from jax.experimental import pallas as pl

_TOKENS = 8192
_HIDDEN = 2048
_EXPERTS = 16
_TB = 1024
_NC = _TOKENS // _TB


def _route_chunk(lg):
    lt = lg.T
    eio = lax.broadcasted_iota(jnp.int32, (_EXPERTS, _TB), 0)
    big_i = jnp.int32(_EXPERTS)
    m1 = jnp.max(lt, axis=0, keepdims=True)
    i1 = jnp.min(jnp.where(lt == m1, eio, big_i), axis=0, keepdims=True)
    sel1 = eio == i1
    ltm = jnp.where(sel1, jnp.float32(-jnp.inf), lt)
    m2 = jnp.max(ltm, axis=0, keepdims=True)
    i2 = jnp.min(jnp.where(ltm == m2, eio, big_i), axis=0, keepdims=True)
    r = jnp.exp(m2 - m1)
    s = r + jnp.float32(1.0)
    w1 = jnp.float32(1.0) / s
    w2 = r / s
    wpair = jnp.concatenate([w1, w2], axis=0).T
    ipair = jnp.concatenate([i1, i2], axis=0).T
    mk1 = sel1.astype(jnp.int32)
    mk2 = (eio == i2).astype(jnp.int32)
    return wpair, ipair, mk1, mk2


def _body(x_ref, w_ref, b_ref, lg_ref, wp_ref, ip_ref, mk_ref):
    i = pl.program_id(0)
    lg = lax.dot_general(
        x_ref[...], w_ref[...],
        (((1,), (1,)), ((), ())),
        preferred_element_type=jnp.float32,
    ) + b_ref[...]
    wpair, ipair, mk1, mk2 = _route_chunk(lg)
    off = i * _TB
    lg_ref[pl.ds(off, _TB), :] = lg
    wp_ref[pl.ds(off, _TB), :] = wpair
    ip_ref[pl.ds(off, _TB), :] = ipair
    mk_ref[:, 0, pl.ds(off, _TB)] = mk1
    mk_ref[:, 1, pl.ds(off, _TB)] = mk2


_fused = pl.pallas_call(
    _body,
    grid=(_NC,),
    in_specs=[
        pl.BlockSpec((_TB, _HIDDEN), lambda i: (i, 0)),
        pl.BlockSpec((_EXPERTS, _HIDDEN), lambda i: (0, 0)),
        pl.BlockSpec((1, _EXPERTS), lambda i: (0, 0)),
    ],
    out_specs=[
        pl.BlockSpec((_TOKENS, _EXPERTS), lambda i: (0, 0)),
        pl.BlockSpec((_TOKENS, 2), lambda i: (0, 0)),
        pl.BlockSpec((_TOKENS, 2), lambda i: (0, 0)),
        pl.BlockSpec((_EXPERTS, 2, _TOKENS), lambda i: (0, 0, 0)),
    ],
    out_shape=[
        jax.ShapeDtypeStruct((_TOKENS, _EXPERTS), jnp.float32),
        jax.ShapeDtypeStruct((_TOKENS, 2), jnp.float32),
        jax.ShapeDtypeStruct((_TOKENS, 2), jnp.int32),
        jax.ShapeDtypeStruct((_EXPERTS, 2, _TOKENS), jnp.int32),
    ],
)


def kernel(x, W, b):
    logits, router_weight, select_idx, expert_mask = _fused(
        x, W, b.reshape(1, _EXPERTS))
    return (logits, router_weight, select_idx, expert_mask)


# speedup vs baseline: 1.0835x; 1.0111x over previous
import jax
import jax.numpy as jnp
from jax import lax
from jax.experimental import pallas as pl

_TOKENS = 8192
_HIDDEN = 2048
_EXPERTS = 16
_TB = 1024


def _body(x_ref, w_ref, b_ref, logits_ref, wpair_ref, ipair_ref, mask_ref):
    lg = lax.dot_general(
        x_ref[...], w_ref[...],
        (((1,), (1,)), ((), ())),
        preferred_element_type=jnp.float32,
    ) + b_ref[...]
    logits_ref[...] = lg

    lt = lg.T
    l0 = lt[0:1, :]
    l1 = lt[1:2, :]
    gt = l1 > l0
    m1 = jnp.where(gt, l1, l0)
    m2 = jnp.where(gt, l0, l1)
    i1 = jnp.where(gt, jnp.int32(1), jnp.int32(0))
    i2 = jnp.where(gt, jnp.int32(0), jnp.int32(1))
    for e in range(2, _EXPERTS):
        le = lt[e:e + 1, :]
        ev = jnp.int32(e)
        gt1 = le > m1
        gt2 = le > m2
        i2 = jnp.where(gt1, i1, jnp.where(gt2, ev, i2))
        m2 = jnp.where(gt1, m1, jnp.where(gt2, le, m2))
        i1 = jnp.where(gt1, ev, i1)
        m1 = jnp.where(gt1, le, m1)

    r = jnp.exp(m2 - m1)
    s = r + jnp.float32(1.0)
    w1 = jnp.float32(1.0) / s
    w2 = r / s

    wpair_ref[...] = jnp.concatenate([w1, w2], axis=0).T
    ipair_ref[...] = jnp.concatenate([i1, i2], axis=0).T

    eio = lax.broadcasted_iota(jnp.int32, (_EXPERTS, _TB), 0)
    mk1 = (eio == i1).astype(jnp.int32)
    mk2 = (eio == i2).astype(jnp.int32)
    mask_ref[...] = jnp.stack([mk1, mk2], axis=1)


_fused = pl.pallas_call(
    _body,
    grid=(_TOKENS // _TB,),
    in_specs=[
        pl.BlockSpec((_TB, _HIDDEN), lambda i: (i, 0)),
        pl.BlockSpec((_EXPERTS, _HIDDEN), lambda i: (0, 0)),
        pl.BlockSpec((1, _EXPERTS), lambda i: (0, 0)),
    ],
    out_specs=[
        pl.BlockSpec((_TB, _EXPERTS), lambda i: (i, 0)),
        pl.BlockSpec((_TB, 2), lambda i: (i, 0)),
        pl.BlockSpec((_TB, 2), lambda i: (i, 0)),
        pl.BlockSpec((_EXPERTS, 2, _TB), lambda i: (0, 0, i)),
    ],
    out_shape=[
        jax.ShapeDtypeStruct((_TOKENS, _EXPERTS), jnp.float32),
        jax.ShapeDtypeStruct((_TOKENS, 2), jnp.float32),
        jax.ShapeDtypeStruct((_TOKENS, 2), jnp.int32),
        jax.ShapeDtypeStruct((_EXPERTS, 2, _TOKENS), jnp.int32),
    ],
)


def kernel(x, W, b):
    logits, router_weight, select_idx, expert_mask = _fused(
        x, W, b.reshape(1, _EXPERTS))
    return (logits, router_weight, select_idx, expert_mask)
